# baseline (speedup 1.0000x reference)
import functools

import jax
import jax.numpy as jnp
from jax import lax
from jax.experimental import pallas as pl
from jax.experimental.pallas import tpu as pltpu
from jax.experimental.pallas import tpu_sc as plsc

N = 10000
E = 320000
D = 128
C = 5
B = 100000

NC, NS = 2, 16
NW = NC * NS
DE = 144
DD = DE - D
PADROWS = 128

ECH = 128
SEGC = 8
NSEG = 10
ENCH = SEGC * NSEG
EPT = ENCH * ECH
E_PAD = EPT * NW

ROWS_PT = N // NS

BCH = 128
BNCH = 25
BPT = BNCH * BCH
B_PAD = BPT * NW


def _sage_aggregate(xe, src4, dst4, zrows):
    mesh = plsc.VectorSubcoreMesh(core_axis_name="c", subcore_axis_name="s")

    @functools.partial(
        pl.kernel,
        out_type=(
            jax.ShapeDtypeStruct((NC, N, D), jnp.float32),
            jax.ShapeDtypeStruct((NC, N, DD), jnp.float32),
        ),
        mesh=mesh,
        scratch_types=[
            pltpu.VMEM((2, SEGC, ECH), jnp.int32),
            pltpu.VMEM((2, SEGC, ECH), jnp.int32),
            pltpu.VMEM((ECH, DE), jnp.float32),
            pltpu.VMEM((ECH, DE), jnp.float32),
            pltpu.VMEM_SHARED((N, DE), jnp.float32),
            pltpu.SemaphoreType.DMA,
            pltpu.SemaphoreType.DMA,
            pltpu.SemaphoreType.DMA,
            pltpu.SemaphoreType.DMA,
            pltpu.SemaphoreType.DMA,
        ],
        compiler_params=pltpu.CompilerParams(use_tc_tiling_on_sc=False),
    )
    def k(xe_hbm, src_hbm, dst_hbm, z_hbm, feat_hbm, deg_hbm, sseg, dseg,
          rows0, rows1, acc, gs0, gs1, ss0, ss1, sg):
        cid = lax.axis_index("c")
        sid = lax.axis_index("s")
        wid = sid * NC + cid
        pltpu.sync_copy(z_hbm, acc.at[pl.ds(sid * ROWS_PT, ROWS_PT)])
        pltpu.sync_copy(src_hbm.at[wid, 0], sseg.at[0])
        pltpu.sync_copy(dst_hbm.at[wid, 0], dseg.at[0])
        pltpu.async_copy(src_hbm.at[wid, 1], sseg.at[1], sg)
        pltpu.async_copy(dst_hbm.at[wid, 1], dseg.at[1], sg)
        plsc.subcore_barrier()

        def gather(sb, l, rows, sem):
            pltpu.async_copy(xe_hbm.at[sseg.at[sb, l]], rows, sem)

        def scatter(sb, l, rows, sem):
            pltpu.async_copy(rows, acc.at[dseg.at[sb, l]], sem, add=True)

        def wait_gather(rows, sem):
            pltpu.make_async_copy(xe_hbm.at[sseg.at[0, 0]], rows, sem).wait()

        def wait_scatter(rows, sem):
            pltpu.make_async_copy(rows, acc.at[dseg.at[0, 0]], sem).wait()

        gather(0, 0, rows0, gs0)
        gather(0, 1, rows1, gs1)

        @pl.loop(0, ENCH // 2 - 1)
        def _(i):
            s_cur = i // 4
            l0 = 2 * (i % 4)
            sb = lax.rem(s_cur, 2)
            s_nxt = (i + 1) // 4
            l2 = 2 * lax.rem(i + 1, 4)
            sb2 = lax.rem(s_nxt, 2)
            wait_gather(rows0, gs0)
            scatter(sb, l0, rows0, ss0)
            wait_gather(rows1, gs1)
            scatter(sb, l0 + 1, rows1, ss1)

            @pl.when(lax.rem(i, 4) == 3)
            def _():
                pltpu.make_async_copy(src_hbm.at[wid, 0], sseg.at[0], sg).wait()
                pltpu.make_async_copy(dst_hbm.at[wid, 0], dseg.at[0], sg).wait()

            wait_scatter(rows0, ss0)
            gather(sb2, l2, rows0, gs0)
            wait_scatter(rows1, ss1)
            gather(sb2, l2 + 1, rows1, gs1)

            @pl.when((lax.rem(i, 4) == 0) & (i > 0) & (s_cur < NSEG - 1))
            def _():
                nb = lax.rem(s_cur + 1, 2)
                pltpu.async_copy(src_hbm.at[wid, s_cur + 1], sseg.at[nb], sg)
                pltpu.async_copy(dst_hbm.at[wid, s_cur + 1], dseg.at[nb], sg)

        sb = lax.rem(NSEG - 1, 2)
        wait_gather(rows0, gs0)
        scatter(sb, SEGC - 2, rows0, ss0)
        wait_gather(rows1, gs1)
        scatter(sb, SEGC - 1, rows1, ss1)
        wait_scatter(rows0, ss0)
        wait_scatter(rows1, ss1)

        plsc.subcore_barrier()
        r0 = sid * ROWS_PT
        pltpu.sync_copy(acc.at[pl.ds(r0, ROWS_PT), pl.ds(0, D)],
                        feat_hbm.at[cid].at[pl.ds(r0, ROWS_PT)])
        pltpu.sync_copy(acc.at[pl.ds(r0, ROWS_PT), pl.ds(D, DD)],
                        deg_hbm.at[cid].at[pl.ds(r0, ROWS_PT)])

    return k(xe, src4, dst4, zrows)


def _encode_body(f_ref, d_ref, x_ref, w_ref, o_ref):
    agg = f_ref[0] + f_ref[1]
    deg = jnp.maximum(d_ref[0, :, 0:1] + d_ref[1, :, 0:1], 1.0)
    mean = agg / deg
    h = (jnp.dot(x_ref[...], w_ref[:D, :], preferred_element_type=jnp.float32)
         + jnp.dot(mean, w_ref[D:, :], preferred_element_type=jnp.float32))
    o_ref[...] = jnp.maximum(h, 0.0)


def _encode(feat, degs, x, W1):
    rb = 2000
    return pl.pallas_call(
        _encode_body,
        grid=(N // rb,),
        in_specs=[
            pl.BlockSpec((NC, rb, D), lambda i: (0, i, 0)),
            pl.BlockSpec((NC, rb, DD), lambda i: (0, i, 0)),
            pl.BlockSpec((rb, D), lambda i: (i, 0)),
            pl.BlockSpec((2 * D, D), lambda i: (0, 0)),
        ],
        out_specs=pl.BlockSpec((rb, D), lambda i: (i, 0)),
        out_shape=jax.ShapeDtypeStruct((N, D), jnp.float32),
    )(feat, degs, x, W1)


def _edge_gather(h, u3, v3):
    mesh = plsc.VectorSubcoreMesh(core_axis_name="c", subcore_axis_name="s")
    out_t = jax.ShapeDtypeStruct((B_PAD, D), jnp.float32)

    @functools.partial(
        pl.kernel,
        out_type=(out_t, out_t),
        mesh=mesh,
        scratch_types=[
            pltpu.VMEM((BNCH, BCH), jnp.int32),
            pltpu.VMEM((BNCH, BCH), jnp.int32),
            pltpu.VMEM((BCH, D), jnp.float32),
            pltpu.VMEM((BCH, D), jnp.float32),
            pltpu.SemaphoreType.DMA,
            pltpu.SemaphoreType.DMA,
            pltpu.SemaphoreType.DMA,
            pltpu.SemaphoreType.DMA,
        ],
        compiler_params=pltpu.CompilerParams(use_tc_tiling_on_sc=False),
    )
    def k(h_hbm, u_hbm, v_hbm, uo_hbm, vo_hbm, idx_u, idx_v, ru, rv,
          gu, gv, wu, wv):
        cid = lax.axis_index("c")
        sid = lax.axis_index("s")
        wid = sid * NC + cid
        base = wid * BPT
        pltpu.sync_copy(u_hbm.at[pl.ds(wid * BNCH, BNCH)], idx_u)
        pltpu.sync_copy(v_hbm.at[pl.ds(wid * BNCH, BNCH)], idx_v)

        def wait_gather(rows, sem):
            pltpu.make_async_copy(h_hbm.at[idx_u.at[0]], rows, sem).wait()

        def wait_write(rows, ohbm, sem):
            pltpu.make_async_copy(rows, ohbm.at[pl.ds(base, BCH)], sem).wait()

        pltpu.async_copy(h_hbm.at[idx_u.at[0]], ru, gu)
        pltpu.async_copy(h_hbm.at[idx_v.at[0]], rv, gv)

        @pl.loop(0, BNCH - 1)
        def _(j):
            off = base + j * BCH
            wait_gather(ru, gu)
            pltpu.async_copy(ru, uo_hbm.at[pl.ds(off, BCH)], wu)
            wait_gather(rv, gv)
            pltpu.async_copy(rv, vo_hbm.at[pl.ds(off, BCH)], wv)
            wait_write(ru, uo_hbm, wu)
            pltpu.async_copy(h_hbm.at[idx_u.at[j + 1]], ru, gu)
            wait_write(rv, vo_hbm, wv)
            pltpu.async_copy(h_hbm.at[idx_v.at[j + 1]], rv, gv)

        off = base + (BNCH - 1) * BCH
        wait_gather(ru, gu)
        pltpu.async_copy(ru, uo_hbm.at[pl.ds(off, BCH)], wu)
        wait_gather(rv, gv)
        pltpu.async_copy(rv, vo_hbm.at[pl.ds(off, BCH)], wv)
        wait_write(ru, uo_hbm, wu)
        wait_write(rv, vo_hbm, wv)

    return k(h, u3, v3)


def _score_body(u_ref, v_ref, w_ref, o_ref):
    e = u_ref[...] * v_ref[...]
    o_ref[...] = lax.dot_general(
        w_ref[...], e, (((0,), (1,)), ((), ())),
        preferred_element_type=jnp.float32)


def _score(u_rows, v_rows, weight):
    rd = 4096
    return pl.pallas_call(
        _score_body,
        grid=((B + rd - 1) // rd,),
        in_specs=[
            pl.BlockSpec((rd, D), lambda i: (i, 0)),
            pl.BlockSpec((rd, D), lambda i: (i, 0)),
            pl.BlockSpec((D, C), lambda i: (0, 0)),
        ],
        out_specs=pl.BlockSpec((C, rd), lambda i: (0, i)),
        out_shape=jax.ShapeDtypeStruct((C, B), jnp.float32),
    )(u_rows, v_rows, weight)


def kernel(x, edge_index, edges, W1, weight):
    f32 = jnp.float32
    x = x.astype(f32)
    W1 = W1.astype(f32)
    weight = weight.astype(f32)
    src = edge_index[0].astype(jnp.int32)
    dst = edge_index[1].astype(jnp.int32)

    xe = jnp.concatenate(
        [x, jnp.ones((N, 1), f32), jnp.zeros((N, DE - D - 1), f32)], axis=1)
    xe = jnp.concatenate([xe, jnp.zeros((PADROWS, DE), f32)], axis=0)

    npad = E_PAD - E
    pad_iota = jnp.arange(npad, dtype=jnp.int32)
    srcp = jnp.concatenate([src, N + (pad_iota % PADROWS)])
    dstp = jnp.concatenate([dst, pad_iota % N])
    src4 = srcp.reshape(NW, NSEG, SEGC, ECH)
    dst4 = dstp.reshape(NW, NSEG, SEGC, ECH)
    zrows = jnp.zeros((ROWS_PT, DE), f32)

    feat, degs = _sage_aggregate(xe, src4, dst4, zrows)
    h = _encode(feat, degs, x, W1)

    edges_b, _ = lax.optimization_barrier((edges, degs))
    u = edges_b[:, 0].astype(jnp.int32)
    v = edges_b[:, 1].astype(jnp.int32)

    bpad = B_PAD - B
    bpad_iota = jnp.arange(bpad, dtype=jnp.int32)
    u3 = jnp.concatenate([u, bpad_iota % N]).reshape(NW * BNCH, BCH)
    v3 = jnp.concatenate([v, bpad_iota % N]).reshape(NW * BNCH, BCH)

    u_rows, v_rows = _edge_gather(h, u3, v3)
    return _score(u_rows, v_rows, weight).T


# speedup vs baseline: 1.0073x; 1.0073x over previous
import functools

import jax
import jax.numpy as jnp
from jax import lax
from jax.experimental import pallas as pl
from jax.experimental.pallas import tpu as pltpu
from jax.experimental.pallas import tpu_sc as plsc

N = 10000
E = 320000
D = 128
C = 5
B = 100000

NC, NS = 2, 16
NW = NC * NS
DE = 144
DD = DE - D
PADROWS = 128

ECH = 128
SEGC = 8
NSEG = 10
ENCH = SEGC * NSEG
EPT = ENCH * ECH
E_PAD = EPT * NW

ROWS_PT = N // NS

BCH = 128
BNCH = 25
BPT = BNCH * BCH
B_PAD = BPT * NW


def _sage_aggregate(xe, src4, dst4, zrows):
    mesh = plsc.VectorSubcoreMesh(core_axis_name="c", subcore_axis_name="s")

    @functools.partial(
        pl.kernel,
        out_type=(
            jax.ShapeDtypeStruct((NC, N, D), jnp.float32),
            jax.ShapeDtypeStruct((NC, N, DD), jnp.float32),
        ),
        mesh=mesh,
        scratch_types=[
            pltpu.VMEM((2, SEGC, ECH), jnp.int32),
            pltpu.VMEM((2, SEGC, ECH), jnp.int32),
            pltpu.VMEM((ECH, DE), jnp.float32),
            pltpu.VMEM((ECH, DE), jnp.float32),
            pltpu.VMEM_SHARED((N, DE), jnp.float32),
            pltpu.SemaphoreType.DMA,
            pltpu.SemaphoreType.DMA,
            pltpu.SemaphoreType.DMA,
            pltpu.SemaphoreType.DMA,
            pltpu.SemaphoreType.DMA,
        ],
        compiler_params=pltpu.CompilerParams(use_tc_tiling_on_sc=False),
    )
    def k(xe_hbm, src_hbm, dst_hbm, z_hbm, feat_hbm, deg_hbm, sseg, dseg,
          rows0, rows1, acc, gs0, gs1, ss0, ss1, sg):
        cid = lax.axis_index("c")
        sid = lax.axis_index("s")
        wid = sid * NC + cid
        pltpu.sync_copy(z_hbm, acc.at[pl.ds(sid * ROWS_PT, ROWS_PT)])
        pltpu.sync_copy(src_hbm.at[wid, 0], sseg.at[0])
        pltpu.sync_copy(dst_hbm.at[wid, 0], dseg.at[0])
        pltpu.async_copy(src_hbm.at[wid, 1], sseg.at[1], sg)
        pltpu.async_copy(dst_hbm.at[wid, 1], dseg.at[1], sg)
        plsc.subcore_barrier()

        def gather(sb, l, rows, sem):
            pltpu.async_copy(xe_hbm.at[sseg.at[sb, l]], rows, sem)

        def scatter(sb, l, rows, sem):
            pltpu.async_copy(rows, acc.at[dseg.at[sb, l]], sem, add=True)

        def wait_gather(rows, sem):
            pltpu.make_async_copy(xe_hbm.at[sseg.at[0, 0]], rows, sem).wait()

        def wait_scatter(rows, sem):
            pltpu.make_async_copy(rows, acc.at[dseg.at[0, 0]], sem).wait()

        gather(0, 0, rows0, gs0)
        gather(0, 1, rows1, gs1)

        @pl.loop(0, ENCH // 2 - 1)
        def _(i):
            s_cur = i // 4
            l0 = 2 * (i % 4)
            sb = lax.rem(s_cur, 2)
            s_nxt = (i + 1) // 4
            l2 = 2 * lax.rem(i + 1, 4)
            sb2 = lax.rem(s_nxt, 2)
            wait_gather(rows0, gs0)
            scatter(sb, l0, rows0, ss0)
            wait_gather(rows1, gs1)
            scatter(sb, l0 + 1, rows1, ss1)

            @pl.when(lax.rem(i, 4) == 3)
            def _():
                pltpu.make_async_copy(src_hbm.at[wid, 0], sseg.at[0], sg).wait()
                pltpu.make_async_copy(dst_hbm.at[wid, 0], dseg.at[0], sg).wait()

            wait_scatter(rows0, ss0)
            gather(sb2, l2, rows0, gs0)
            wait_scatter(rows1, ss1)
            gather(sb2, l2 + 1, rows1, gs1)

            @pl.when((lax.rem(i, 4) == 0) & (i > 0) & (s_cur < NSEG - 1))
            def _():
                nb = lax.rem(s_cur + 1, 2)
                pltpu.async_copy(src_hbm.at[wid, s_cur + 1], sseg.at[nb], sg)
                pltpu.async_copy(dst_hbm.at[wid, s_cur + 1], dseg.at[nb], sg)

        sb = lax.rem(NSEG - 1, 2)
        wait_gather(rows0, gs0)
        scatter(sb, SEGC - 2, rows0, ss0)
        wait_gather(rows1, gs1)
        scatter(sb, SEGC - 1, rows1, ss1)
        wait_scatter(rows0, ss0)
        wait_scatter(rows1, ss1)

        plsc.subcore_barrier()
        r0 = sid * ROWS_PT
        pltpu.sync_copy(acc.at[pl.ds(r0, ROWS_PT), pl.ds(0, D)],
                        feat_hbm.at[cid].at[pl.ds(r0, ROWS_PT)])
        pltpu.sync_copy(acc.at[pl.ds(r0, ROWS_PT), pl.ds(D, DD)],
                        deg_hbm.at[cid].at[pl.ds(r0, ROWS_PT)])

    return k(xe, src4, dst4, zrows)


def _encode_body(f_ref, d_ref, x_ref, w_ref, o_ref):
    agg = f_ref[0] + f_ref[1]
    deg = jnp.maximum(d_ref[0, :, 0:1] + d_ref[1, :, 0:1], 1.0)
    mean = agg / deg
    h = (jnp.dot(x_ref[...], w_ref[:D, :], preferred_element_type=jnp.float32)
         + jnp.dot(mean, w_ref[D:, :], preferred_element_type=jnp.float32))
    o_ref[...] = jnp.maximum(h, 0.0)


def _encode(feat, degs, x, W1):
    rb = 2000
    return pl.pallas_call(
        _encode_body,
        grid=(N // rb,),
        in_specs=[
            pl.BlockSpec((NC, rb, D), lambda i: (0, i, 0)),
            pl.BlockSpec((NC, rb, DD), lambda i: (0, i, 0)),
            pl.BlockSpec((rb, D), lambda i: (i, 0)),
            pl.BlockSpec((2 * D, D), lambda i: (0, 0)),
        ],
        out_specs=pl.BlockSpec((rb, D), lambda i: (i, 0)),
        out_shape=jax.ShapeDtypeStruct((N, D), jnp.float32),
    )(feat, degs, x, W1)


def _edge_gather(h, u3, v3):
    mesh = plsc.VectorSubcoreMesh(core_axis_name="c", subcore_axis_name="s")
    out_t = jax.ShapeDtypeStruct((B_PAD, D), jnp.float32)

    @functools.partial(
        pl.kernel,
        out_type=(out_t, out_t),
        mesh=mesh,
        scratch_types=[
            pltpu.VMEM((BNCH, BCH), jnp.int32),
            pltpu.VMEM((BNCH, BCH), jnp.int32),
            pltpu.VMEM((BCH, D), jnp.float32),
            pltpu.VMEM((BCH, D), jnp.float32),
            pltpu.SemaphoreType.DMA,
            pltpu.SemaphoreType.DMA,
            pltpu.SemaphoreType.DMA,
            pltpu.SemaphoreType.DMA,
        ],
        compiler_params=pltpu.CompilerParams(use_tc_tiling_on_sc=False),
    )
    def k(h_hbm, u_hbm, v_hbm, uo_hbm, vo_hbm, idx_u, idx_v, ru, rv,
          gu, gv, wu, wv):
        cid = lax.axis_index("c")
        sid = lax.axis_index("s")
        wid = sid * NC + cid
        base = wid * BPT
        pltpu.sync_copy(u_hbm.at[pl.ds(wid * BNCH, BNCH)], idx_u)
        pltpu.sync_copy(v_hbm.at[pl.ds(wid * BNCH, BNCH)], idx_v)

        def wait_gather(rows, sem):
            pltpu.make_async_copy(h_hbm.at[idx_u.at[0]], rows, sem).wait()

        def wait_write(rows, ohbm, sem):
            pltpu.make_async_copy(rows, ohbm.at[pl.ds(base, BCH)], sem).wait()

        pltpu.async_copy(h_hbm.at[idx_u.at[0]], ru, gu)
        pltpu.async_copy(h_hbm.at[idx_v.at[0]], rv, gv)

        @pl.loop(0, BNCH - 1)
        def _(j):
            off = base + j * BCH
            wait_gather(ru, gu)
            pltpu.async_copy(ru, uo_hbm.at[pl.ds(off, BCH)], wu)
            wait_gather(rv, gv)
            pltpu.async_copy(rv, vo_hbm.at[pl.ds(off, BCH)], wv)
            wait_write(ru, uo_hbm, wu)
            pltpu.async_copy(h_hbm.at[idx_u.at[j + 1]], ru, gu)
            wait_write(rv, vo_hbm, wv)
            pltpu.async_copy(h_hbm.at[idx_v.at[j + 1]], rv, gv)

        off = base + (BNCH - 1) * BCH
        wait_gather(ru, gu)
        pltpu.async_copy(ru, uo_hbm.at[pl.ds(off, BCH)], wu)
        wait_gather(rv, gv)
        pltpu.async_copy(rv, vo_hbm.at[pl.ds(off, BCH)], wv)
        wait_write(ru, uo_hbm, wu)
        wait_write(rv, vo_hbm, wv)

    return k(h, u3, v3)


def _score_body(u_ref, v_ref, w_ref, o_ref):
    e = u_ref[...] * v_ref[...]
    o_ref[...] = lax.dot_general(
        w_ref[...], e, (((0,), (1,)), ((), ())),
        preferred_element_type=jnp.float32)


def _score(u_rows, v_rows, weight):
    rd = 4096
    return pl.pallas_call(
        _score_body,
        grid=((B + rd - 1) // rd,),
        in_specs=[
            pl.BlockSpec((rd, D), lambda i: (i, 0)),
            pl.BlockSpec((rd, D), lambda i: (i, 0)),
            pl.BlockSpec((D, C), lambda i: (0, 0)),
        ],
        out_specs=pl.BlockSpec((C, rd), lambda i: (0, i)),
        out_shape=jax.ShapeDtypeStruct((C, B), jnp.float32),
    )(u_rows, v_rows, weight)


def kernel(x, edge_index, edges, W1, weight):
    f32 = jnp.float32
    x = x.astype(f32)
    W1 = W1.astype(f32)
    weight = weight.astype(f32)
    src = edge_index[0].astype(jnp.int32)
    dst = edge_index[1].astype(jnp.int32)

    xe = jnp.concatenate(
        [x, jnp.ones((N, 1), f32), jnp.zeros((N, DE - D - 1), f32)], axis=1)
    xe = jnp.concatenate([xe, jnp.zeros((PADROWS, DE), f32)], axis=0)

    npad = E_PAD - E
    pad_iota = jnp.arange(npad, dtype=jnp.int32)
    srcp = jnp.concatenate([src, N + (pad_iota % PADROWS)])
    dstp = jnp.concatenate([dst, pad_iota % N])
    src4 = srcp.reshape(NW, NSEG, SEGC, ECH)
    dst4 = dstp.reshape(NW, NSEG, SEGC, ECH)
    zrows = jnp.zeros((ROWS_PT, DE), f32)

    feat, degs = _sage_aggregate(xe, src4, dst4, zrows)
    h = _encode(feat, degs, x, W1)

    u = edges[:, 0].astype(jnp.int32)
    v = edges[:, 1].astype(jnp.int32)

    bpad = B_PAD - B
    bpad_iota = jnp.arange(bpad, dtype=jnp.int32)
    u3 = jnp.concatenate([u, bpad_iota % N]).reshape(NW * BNCH, BCH)
    v3 = jnp.concatenate([v, bpad_iota % N]).reshape(NW * BNCH, BCH)

    u_rows, v_rows = _edge_gather(h, u3, v3)
    return _score(u_rows, v_rows, weight).T
